# baseline (speedup 1.0000x reference)
import jax
import jax.numpy as jnp
from jax import lax
from jax.experimental import pallas as pl
from jax.experimental.pallas import tpu as pltpu
from jax.experimental.pallas import tpu_sc as plsc

B = 16384
K = 50
D = 64
V = 1000000
NC = 2
NS = 16
NW = NC * NS
SENT_PER_W = B // NW
S = 16
CHUNKS = SENT_PER_W // S
ROWS = S * K
LANES = 16
DV = D // LANES
KP = 64
KG = (K + LANES - 1) // LANES
BT = 2048


def _tr_body(tT_ref, out_ref):
    xt = lax.transpose(tT_ref[...], (1, 0))
    out_ref[:, 0:D] = xt
    out_ref[:, D:2 * D] = xt


def _sc_body(ids_hbm, w_hbm, table_hbm, out_hbm, idx_v, w_v, rows_v, out_v, sem):
    wid = lax.axis_index("s") * NC + lax.axis_index("c")
    base_s = wid * SENT_PER_W

    def chunk_body(c, carry):
        s0 = pl.multiple_of(base_s + c * S, S)
        f0 = pl.multiple_of(s0 * K, S * K)
        p0 = pl.multiple_of(s0 * KP, S * KP)
        pltpu.sync_copy(ids_hbm.at[pl.ds(f0, ROWS)], idx_v)
        pltpu.sync_copy(w_hbm.at[pl.ds(p0, S * KP)], w_v)
        pltpu.async_copy(table_hbm.at[idx_v], rows_v, sem).wait()

        def sent_body(s, carry2):
            r0 = s * K
            wb = s * KP
            zero = jnp.zeros((LANES,), jnp.float32)
            accs = [zero] * DV
            for g in range(KG):
                cnt = min(LANES, K - g * LANES)
                w16 = w_v[pl.ds(wb + g * LANES, LANES)]
                for j in range(cnt):
                    wv = lax.broadcast(w16[j], (LANES,))
                    fi = r0 + g * LANES + j
                    for d in range(DV):
                        accs[d] = accs[d] + wv * rows_v[fi, pl.ds(d * LANES, LANES)]
            inv_k = jnp.float32(1.0 / K)
            orow = lax.shift_right_logical(s, 1)
            obase = lax.shift_left(lax.bitwise_and(s, 1), 6)
            for d in range(DV):
                out_v[orow, pl.ds(obase + d * LANES, LANES)] = accs[d] * inv_k
            return carry2

        lax.fori_loop(0, S, sent_body, 0)
        pltpu.sync_copy(out_v, out_hbm.at[pl.ds(pl.multiple_of(s0 // 2, S // 2), S // 2)])
        return carry

    lax.fori_loop(0, CHUNKS, chunk_body, 0)


@jax.jit
def kernel(token_ids, weights, table):
    ids_flat = token_ids.astype(jnp.int32).reshape(-1)
    w_flat = jnp.pad(weights, ((0, 0), (0, KP - K))).reshape(-1)

    table_t = table.T
    n_blk = (V + BT - 1) // BT
    table_pad = pl.pallas_call(
        _tr_body,
        grid=(n_blk,),
        in_specs=[pl.BlockSpec((D, BT), lambda i: (0, i))],
        out_specs=pl.BlockSpec((BT, 2 * D), lambda i: (i, 0)),
        out_shape=jax.ShapeDtypeStruct((V, 2 * D), jnp.float32),
    )(table_t)

    mesh = plsc.VectorSubcoreMesh(core_axis_name="c", subcore_axis_name="s")
    out2 = pl.kernel(
        _sc_body,
        out_type=jax.ShapeDtypeStruct((B // 2, 2 * D), jnp.float32),
        mesh=mesh,
        scratch_types=[
            pltpu.VMEM((ROWS,), jnp.int32),
            pltpu.VMEM((S * KP,), jnp.float32),
            pltpu.VMEM((ROWS, 2 * D), jnp.float32),
            pltpu.VMEM((S // 2, 2 * D), jnp.float32),
            pltpu.SemaphoreType.DMA,
        ],
    )(ids_flat, w_flat, table_pad)
    return out2.reshape(B, D)


# speedup vs baseline: 1.0688x; 1.0688x over previous
import jax
import jax.numpy as jnp
from jax import lax
from jax.experimental import pallas as pl
from jax.experimental.pallas import tpu as pltpu
from jax.experimental.pallas import tpu_sc as plsc

B = 16384
K = 50
D = 64
V = 1000000
NC = 2
NS = 16
NW = NC * NS
SENT_PER_W = B // NW
S = 16
CHUNKS = SENT_PER_W // S
ROWS = S * K
LANES = 16
DV = D // LANES
KP = 64
KG = (K + LANES - 1) // LANES
BT = 2048


def _tr_body(tT_ref, out_ref):
    x = tT_ref[...]
    xc = jnp.concatenate([x, x], axis=0)
    out_ref[...] = lax.transpose(xc, (1, 0))


def _sc_body(ids_hbm, w_hbm, table_hbm, out_hbm, idx_v, w_v, rows_v, out_v, sem):
    wid = lax.axis_index("s") * NC + lax.axis_index("c")
    base_s = wid * SENT_PER_W

    def chunk_body(c, carry):
        s0 = pl.multiple_of(base_s + c * S, S)
        f0 = pl.multiple_of(s0 * K, S * K)
        p0 = pl.multiple_of(s0 * KP, S * KP)
        pltpu.sync_copy(ids_hbm.at[pl.ds(f0, ROWS)], idx_v)
        pltpu.sync_copy(w_hbm.at[pl.ds(p0, S * KP)], w_v)
        pltpu.async_copy(table_hbm.at[idx_v], rows_v, sem).wait()

        def sent_body(s, carry2):
            r0 = s * K
            wb = s * KP
            zero = jnp.zeros((LANES,), jnp.float32)
            accs = [zero] * DV
            for g in range(KG):
                cnt = min(LANES, K - g * LANES)
                w16 = w_v[pl.ds(wb + g * LANES, LANES)]
                for j in range(cnt):
                    wv = lax.broadcast(w16[j], (LANES,))
                    fi = r0 + g * LANES + j
                    for d in range(DV):
                        accs[d] = accs[d] + wv * rows_v[fi, pl.ds(d * LANES, LANES)]
            inv_k = jnp.float32(1.0 / K)
            orow = lax.shift_right_logical(s, 1)
            obase = lax.shift_left(lax.bitwise_and(s, 1), 6)
            for d in range(DV):
                out_v[orow, pl.ds(obase + d * LANES, LANES)] = accs[d] * inv_k
            return carry2

        lax.fori_loop(0, S, sent_body, 0)
        pltpu.sync_copy(out_v, out_hbm.at[pl.ds(pl.multiple_of(s0 // 2, S // 2), S // 2)])
        return carry

    lax.fori_loop(0, CHUNKS, chunk_body, 0)


@jax.jit
def kernel(token_ids, weights, table):
    ids_flat = token_ids.astype(jnp.int32).reshape(-1)
    w_flat = jnp.pad(weights, ((0, 0), (0, KP - K))).reshape(-1)

    table_t = table.T
    n_blk = (V + BT - 1) // BT
    table_pad = pl.pallas_call(
        _tr_body,
        grid=(n_blk,),
        in_specs=[pl.BlockSpec((D, BT), lambda i: (0, i))],
        out_specs=pl.BlockSpec((BT, 2 * D), lambda i: (i, 0)),
        out_shape=jax.ShapeDtypeStruct((V, 2 * D), jnp.float32),
    )(table_t)

    mesh = plsc.VectorSubcoreMesh(core_axis_name="c", subcore_axis_name="s")
    out2 = pl.kernel(
        _sc_body,
        out_type=jax.ShapeDtypeStruct((B // 2, 2 * D), jnp.float32),
        mesh=mesh,
        scratch_types=[
            pltpu.VMEM((ROWS,), jnp.int32),
            pltpu.VMEM((S * KP,), jnp.float32),
            pltpu.VMEM((ROWS, 2 * D), jnp.float32),
            pltpu.VMEM((S // 2, 2 * D), jnp.float32),
            pltpu.SemaphoreType.DMA,
        ],
    )(ids_flat, w_flat, table_pad)
    return out2.reshape(B, D)


# speedup vs baseline: 1.4274x; 1.3355x over previous
import jax
import jax.numpy as jnp
from jax import lax
from jax.experimental import pallas as pl
from jax.experimental.pallas import tpu as pltpu
from jax.experimental.pallas import tpu_sc as plsc

B = 16384
K = 50
D = 64
V = 1000000
NC = 2
NS = 16
NW = NC * NS
SENT_PER_W = B // NW
S = 16
CHUNKS = SENT_PER_W // S
ROWS = S * K
LANES = 16
DV = D // LANES
KP = 64
KG = (K + LANES - 1) // LANES
BT = 8192


def _tr_body(tT_ref, out_ref):
    x = tT_ref[...]
    xc = jnp.concatenate([x, x], axis=0)
    out_ref[...] = lax.transpose(xc, (1, 0))


def _sc_body(ids_hbm, w_hbm, table_hbm, out_hbm, idx_v, w_v, rows_v, out_v, sem):
    wid = lax.axis_index("s") * NC + lax.axis_index("c")
    base_s = wid * SENT_PER_W

    def chunk_body(c, carry):
        s0 = pl.multiple_of(base_s + c * S, S)
        f0 = pl.multiple_of(s0 * K, S * K)
        p0 = pl.multiple_of(s0 * KP, S * KP)
        pltpu.sync_copy(ids_hbm.at[pl.ds(f0, ROWS)], idx_v)
        pltpu.sync_copy(w_hbm.at[pl.ds(p0, S * KP)], w_v)
        pltpu.async_copy(table_hbm.at[idx_v], rows_v, sem).wait()

        def sent_body(s, carry2):
            r0 = s * K
            wb = s * KP
            zero = jnp.zeros((LANES,), jnp.float32)
            accs = [zero] * DV
            for g in range(KG):
                cnt = min(LANES, K - g * LANES)
                w16 = w_v[pl.ds(wb + g * LANES, LANES)]
                for j in range(cnt):
                    wv = lax.broadcast(w16[j], (LANES,))
                    fi = r0 + g * LANES + j
                    for d in range(DV):
                        accs[d] = accs[d] + wv * rows_v[fi, pl.ds(d * LANES, LANES)]
            inv_k = jnp.float32(1.0 / K)
            orow = lax.shift_right_logical(s, 1)
            obase = lax.shift_left(lax.bitwise_and(s, 1), 6)
            for d in range(DV):
                out_v[orow, pl.ds(obase + d * LANES, LANES)] = accs[d] * inv_k
            return carry2

        lax.fori_loop(0, S, sent_body, 0)
        pltpu.sync_copy(out_v, out_hbm.at[pl.ds(pl.multiple_of(s0 // 2, S // 2), S // 2)])
        return carry

    lax.fori_loop(0, CHUNKS, chunk_body, 0)


@jax.jit
def kernel(token_ids, weights, table):
    ids_flat = token_ids.astype(jnp.int32).reshape(-1)
    w_flat = jnp.pad(weights, ((0, 0), (0, KP - K))).reshape(-1)

    table_t = table.T
    n_blk = (V + BT - 1) // BT
    table_pad = pl.pallas_call(
        _tr_body,
        grid=(n_blk,),
        in_specs=[pl.BlockSpec((D, BT), lambda i: (0, i))],
        out_specs=pl.BlockSpec((BT, 2 * D), lambda i: (i, 0)),
        out_shape=jax.ShapeDtypeStruct((V, 2 * D), jnp.float32),
    )(table_t)

    mesh = plsc.VectorSubcoreMesh(core_axis_name="c", subcore_axis_name="s")
    out2 = pl.kernel(
        _sc_body,
        out_type=jax.ShapeDtypeStruct((B // 2, 2 * D), jnp.float32),
        mesh=mesh,
        scratch_types=[
            pltpu.VMEM((ROWS,), jnp.int32),
            pltpu.VMEM((S * KP,), jnp.float32),
            pltpu.VMEM((ROWS, 2 * D), jnp.float32),
            pltpu.VMEM((S // 2, 2 * D), jnp.float32),
            pltpu.SemaphoreType.DMA,
        ],
    )(ids_flat, w_flat, table_pad)
    return out2.reshape(B, D)


# speedup vs baseline: 1.4786x; 1.0359x over previous
import jax
import jax.numpy as jnp
from jax import lax
from jax.experimental import pallas as pl
from jax.experimental.pallas import tpu as pltpu
from jax.experimental.pallas import tpu_sc as plsc

B = 16384
K = 50
D = 64
V = 1000000
NC = 2
NS = 16
NW = NC * NS
SENT_PER_W = B // NW
S = 16
CHUNKS = SENT_PER_W // S
ROWS = S * K
LANES = 16
DV = D // LANES
KP = 64
KG = (K + LANES - 1) // LANES
BT = 16384


def _tr_body(tT_ref, out_ref):
    x = tT_ref[...]
    xc = jnp.concatenate([x, x], axis=0)
    out_ref[...] = lax.transpose(xc, (1, 0))


def _sc_body(ids_hbm, w_hbm, table_hbm, out_hbm, idx_v, w_v, rows_v, out_v, sem):
    wid = lax.axis_index("s") * NC + lax.axis_index("c")
    base_s = wid * SENT_PER_W

    def chunk_body(c, carry):
        s0 = pl.multiple_of(base_s + c * S, S)
        f0 = pl.multiple_of(s0 * K, S * K)
        p0 = pl.multiple_of(s0 * KP, S * KP)
        pltpu.sync_copy(ids_hbm.at[pl.ds(f0, ROWS)], idx_v)
        pltpu.sync_copy(w_hbm.at[pl.ds(p0, S * KP)], w_v)
        pltpu.async_copy(table_hbm.at[idx_v], rows_v, sem).wait()

        def sent_body(s, carry2):
            r0 = s * K
            wb = s * KP
            zero = jnp.zeros((LANES,), jnp.float32)
            accs = [zero] * DV
            for g in range(KG):
                cnt = min(LANES, K - g * LANES)
                w16 = w_v[pl.ds(wb + g * LANES, LANES)]
                for j in range(cnt):
                    wv = lax.broadcast(w16[j], (LANES,))
                    fi = r0 + g * LANES + j
                    for d in range(DV):
                        accs[d] = accs[d] + wv * rows_v[fi, pl.ds(d * LANES, LANES)]
            inv_k = jnp.float32(1.0 / K)
            orow = lax.shift_right_logical(s, 1)
            obase = lax.shift_left(lax.bitwise_and(s, 1), 6)
            for d in range(DV):
                out_v[orow, pl.ds(obase + d * LANES, LANES)] = accs[d] * inv_k
            return carry2

        lax.fori_loop(0, S, sent_body, 0)
        pltpu.sync_copy(out_v, out_hbm.at[pl.ds(pl.multiple_of(s0 // 2, S // 2), S // 2)])
        return carry

    lax.fori_loop(0, CHUNKS, chunk_body, 0)


@jax.jit
def kernel(token_ids, weights, table):
    ids_flat = token_ids.astype(jnp.int32).reshape(-1)
    w_flat = jnp.pad(weights, ((0, 0), (0, KP - K))).reshape(-1)

    table_t = table.T
    n_blk = (V + BT - 1) // BT
    table_pad = pl.pallas_call(
        _tr_body,
        grid=(n_blk,),
        in_specs=[pl.BlockSpec((D, BT), lambda i: (0, i))],
        out_specs=pl.BlockSpec((BT, 2 * D), lambda i: (i, 0)),
        out_shape=jax.ShapeDtypeStruct((V, 2 * D), jnp.float32),
    )(table_t)

    mesh = plsc.VectorSubcoreMesh(core_axis_name="c", subcore_axis_name="s")
    out2 = pl.kernel(
        _sc_body,
        out_type=jax.ShapeDtypeStruct((B // 2, 2 * D), jnp.float32),
        mesh=mesh,
        scratch_types=[
            pltpu.VMEM((ROWS,), jnp.int32),
            pltpu.VMEM((S * KP,), jnp.float32),
            pltpu.VMEM((ROWS, 2 * D), jnp.float32),
            pltpu.VMEM((S // 2, 2 * D), jnp.float32),
            pltpu.SemaphoreType.DMA,
        ],
    )(ids_flat, w_flat, table_pad)
    return out2.reshape(B, D)


# speedup vs baseline: 1.7149x; 1.1598x over previous
import jax
import jax.numpy as jnp
from jax import lax
from jax.experimental import pallas as pl
from jax.experimental.pallas import tpu as pltpu
from jax.experimental.pallas import tpu_sc as plsc

B = 16384
K = 50
D = 64
V = 1000000
NC = 2
NS = 16
NW = NC * NS
SENT_PER_W = B // NW
S = 16
CHUNKS = SENT_PER_W // S
ROWS = S * K
LANES = 16
DV = D // LANES
KP = 64
KG = (K + LANES - 1) // LANES
OFF = 1 << 18
BT = 8192
NBQ = OFF // BT


def _tr_body(t0_ref, t1_ref, t2_ref, t3_ref, out_ref):
    parts = []
    half = jnp.uint32(0x8000)
    himask = jnp.uint32(0xFFFF0000)
    for tq in (t0_ref, t1_ref, t2_ref, t3_ref):
        bits = lax.bitcast_convert_type(tq[...], jnp.uint32)
        lo = lax.shift_right_logical(bits[0:D // 2, :] + half, jnp.uint32(16))
        hi = lax.bitwise_and(bits[D // 2:D, :] + half, himask)
        parts.append(lax.bitwise_or(lo, hi))
    xc = jnp.concatenate(parts, axis=0)
    out_ref[...] = lax.bitcast_convert_type(
        lax.transpose(xc, (1, 0)), jnp.int32)


def _sc_body(ids_hbm, idsp_hbm, w_hbm, table_hbm, out_hbm,
             idx_v, gidx_v, ip_v, w_v, rows_v, out_v, sem):
    wid = lax.axis_index("s") * NC + lax.axis_index("c")
    base_s = wid * SENT_PER_W

    def chunk_body(c, carry):
        s0 = pl.multiple_of(base_s + c * S, S)
        f0 = pl.multiple_of(s0 * K, S * K)
        p0 = pl.multiple_of(s0 * KP, S * KP)
        pltpu.sync_copy(ids_hbm.at[pl.ds(f0, ROWS)], idx_v)
        pltpu.sync_copy(idsp_hbm.at[pl.ds(p0, S * KP)], ip_v)
        pltpu.sync_copy(w_hbm.at[pl.ds(p0, S * KP)], w_v)
        row_mask = jnp.full((LANES,), OFF - 1, jnp.int32)
        for g in range(ROWS // LANES):
            sl = pl.ds(g * LANES, LANES)
            gidx_v[sl] = lax.bitwise_and(idx_v[sl], row_mask)
        pltpu.async_copy(table_hbm.at[gidx_v], rows_v, sem).wait()

        himask = jnp.full((LANES,), -65536, jnp.int32)

        def sent_body(s, carry2):
            r0 = s * K
            wb = s * KP
            zero = jnp.zeros((LANES,), jnp.float32)
            accs = [zero] * DV
            for g in range(KG):
                cnt = min(LANES, K - g * LANES)
                sl = pl.ds(wb + g * LANES, LANES)
                w16 = w_v[sl]
                o16 = lax.shift_left(lax.shift_right_logical(ip_v[sl], 18), 5)
                for j in range(cnt):
                    wv = lax.broadcast(w16[j], (LANES,))
                    off = o16[j]
                    fi = r0 + g * LANES + j
                    for h in range(D // 32):
                        xi = rows_v[fi, pl.ds(off + h * LANES, LANES)]
                        lo = lax.bitcast_convert_type(
                            lax.shift_left(xi, 16), jnp.float32)
                        hi = lax.bitcast_convert_type(
                            lax.bitwise_and(xi, himask), jnp.float32)
                        accs[h] = accs[h] + wv * lo
                        accs[2 + h] = accs[2 + h] + wv * hi
            inv_k = jnp.float32(1.0 / K)
            orow = lax.shift_right_logical(s, 1)
            obase = lax.shift_left(lax.bitwise_and(s, 1), 6)
            for d in range(DV):
                out_v[orow, pl.ds(obase + d * LANES, LANES)] = accs[d] * inv_k
            return carry2

        lax.fori_loop(0, S, sent_body, 0)
        pltpu.sync_copy(out_v, out_hbm.at[pl.ds(pl.multiple_of(s0 // 2, S // 2), S // 2)])
        return carry

    lax.fori_loop(0, CHUNKS, chunk_body, 0)


@jax.jit
def kernel(token_ids, weights, table):
    ids = token_ids.astype(jnp.int32)
    ids_flat = ids.reshape(-1)
    ids_pad = jnp.pad(ids, ((0, 0), (0, KP - K))).reshape(-1)
    w_flat = jnp.pad(weights, ((0, 0), (0, KP - K))).reshape(-1)

    table_t = table.T
    last_blk = (V + BT - 1) // BT - 1
    in_specs = [
        pl.BlockSpec((D, BT), lambda i, q=q: (0, jnp.minimum(i + q * NBQ, last_blk)))
        for q in range(4)
    ]
    table_pk = pl.pallas_call(
        _tr_body,
        grid=(NBQ,),
        in_specs=in_specs,
        out_specs=pl.BlockSpec((BT, 2 * D), lambda i: (i, 0)),
        out_shape=jax.ShapeDtypeStruct((OFF, 2 * D), jnp.int32),
    )(table_t, table_t, table_t, table_t)

    mesh = plsc.VectorSubcoreMesh(core_axis_name="c", subcore_axis_name="s")
    out2 = pl.kernel(
        _sc_body,
        out_type=jax.ShapeDtypeStruct((B // 2, 2 * D), jnp.float32),
        mesh=mesh,
        scratch_types=[
            pltpu.VMEM((ROWS,), jnp.int32),
            pltpu.VMEM((ROWS,), jnp.int32),
            pltpu.VMEM((S * KP,), jnp.int32),
            pltpu.VMEM((S * KP,), jnp.float32),
            pltpu.VMEM((ROWS, 2 * D), jnp.int32),
            pltpu.VMEM((S // 2, 2 * D), jnp.float32),
            pltpu.SemaphoreType.DMA,
        ],
    )(ids_flat, ids_pad, w_flat, table_pk)
    return out2.reshape(B, D)


# speedup vs baseline: 2.0035x; 1.1683x over previous
import jax
import jax.numpy as jnp
from jax import lax
from jax.experimental import pallas as pl
from jax.experimental.pallas import tpu as pltpu
from jax.experimental.pallas import tpu_sc as plsc

B = 16384
K = 50
D = 64
V = 1000000
NC = 2
NS = 16
NW = NC * NS
SENT_PER_W = B // NW
S = 8
CHUNKS = SENT_PER_W // S
ROWS = S * K
LANES = 16
DV = D // LANES
KP = 64
KG = (K + LANES - 1) // LANES
OFF = 1 << 18
BT = 8192
NBQ = OFF // BT


def _tr_body(t0_ref, t1_ref, t2_ref, t3_ref, out_ref):
    parts = []
    half = jnp.uint32(0x8000)
    himask = jnp.uint32(0xFFFF0000)
    for tq in (t0_ref, t1_ref, t2_ref, t3_ref):
        bits = lax.bitcast_convert_type(tq[...], jnp.uint32)
        lo = lax.shift_right_logical(bits[0:D // 2, :] + half, jnp.uint32(16))
        hi = lax.bitwise_and(bits[D // 2:D, :] + half, himask)
        parts.append(lax.bitwise_or(lo, hi))
    xc = jnp.concatenate(parts, axis=0)
    out_ref[...] = lax.bitcast_convert_type(
        lax.transpose(xc, (1, 0)), jnp.int32)


def _sc_body(ids_hbm, idsp_hbm, w_hbm, table_hbm, out_hbm,
             idxa, idxb, ga, gb, ipa, ipb, wva, wvb, ra, rb, oa, ob,
             sema, semb):
    wid = lax.axis_index("s") * NC + lax.axis_index("c")
    base_s = wid * SENT_PER_W
    sets = ((idxa, ga, ipa, wva, ra, oa, sema),
            (idxb, gb, ipb, wvb, rb, ob, semb))
    row_mask = jnp.full((LANES,), OFF - 1, jnp.int32)
    himask = jnp.full((LANES,), -65536, jnp.int32)

    def prep(c, st):
        idx_v, gidx_v, ip_v, w_v, rows_v, out_v, sem = st
        s0 = pl.multiple_of(base_s + c * S, S)
        f0 = pl.multiple_of(s0 * K, S * K)
        p0 = pl.multiple_of(s0 * KP, S * KP)
        pltpu.sync_copy(ids_hbm.at[pl.ds(f0, ROWS)], idx_v)
        pltpu.sync_copy(idsp_hbm.at[pl.ds(p0, S * KP)], ip_v)
        pltpu.sync_copy(w_hbm.at[pl.ds(p0, S * KP)], w_v)
        for g in range(ROWS // LANES):
            sl = pl.ds(g * LANES, LANES)
            gidx_v[sl] = lax.bitwise_and(idx_v[sl], row_mask)
        pltpu.async_copy(table_hbm.at[gidx_v], rows_v, sem)

    def waitg(st):
        idx_v, gidx_v, ip_v, w_v, rows_v, out_v, sem = st
        pltpu.make_async_copy(table_hbm.at[gidx_v], rows_v, sem).wait()

    def compute(c, st):
        idx_v, gidx_v, ip_v, w_v, rows_v, out_v, sem = st
        s0 = pl.multiple_of(base_s + c * S, S)

        def sent_body(s, carry2):
            r0 = s * K
            wb = s * KP
            zero = jnp.zeros((LANES,), jnp.float32)
            accs = [zero] * DV
            for g in range(KG):
                cnt = min(LANES, K - g * LANES)
                sl = pl.ds(wb + g * LANES, LANES)
                w16 = w_v[sl]
                o16 = lax.shift_left(lax.shift_right_logical(ip_v[sl], 18), 5)
                for j in range(cnt):
                    wv = lax.broadcast(w16[j], (LANES,))
                    off = o16[j]
                    fi = r0 + g * LANES + j
                    for h in range(D // 32):
                        xi = rows_v[fi, pl.ds(off + h * LANES, LANES)]
                        lo = lax.bitcast_convert_type(
                            lax.shift_left(xi, 16), jnp.float32)
                        hi = lax.bitcast_convert_type(
                            lax.bitwise_and(xi, himask), jnp.float32)
                        accs[h] = accs[h] + wv * lo
                        accs[2 + h] = accs[2 + h] + wv * hi
            inv_k = jnp.float32(1.0 / K)
            orow = lax.shift_right_logical(s, 1)
            obase = lax.shift_left(lax.bitwise_and(s, 1), 6)
            for d in range(DV):
                out_v[orow, pl.ds(obase + d * LANES, LANES)] = accs[d] * inv_k
            return carry2

        lax.fori_loop(0, S, sent_body, 0)
        pltpu.sync_copy(out_v, out_hbm.at[pl.ds(pl.multiple_of(s0 // 2, S // 2), S // 2)])

    prep(0, sets[0])

    def pair_body(i, carry):
        prep(2 * i + 1, sets[1])
        waitg(sets[0])
        compute(2 * i, sets[0])

        @pl.when(i < CHUNKS // 2 - 1)
        def _():
            prep(2 * i + 2, sets[0])

        waitg(sets[1])
        compute(2 * i + 1, sets[1])
        return carry

    lax.fori_loop(0, CHUNKS // 2, pair_body, 0)


@jax.jit
def kernel(token_ids, weights, table):
    ids = token_ids.astype(jnp.int32)
    ids_flat = ids.reshape(-1)
    ids_pad = jnp.pad(ids, ((0, 0), (0, KP - K))).reshape(-1)
    w_flat = jnp.pad(weights, ((0, 0), (0, KP - K))).reshape(-1)

    table_t = table.T
    last_blk = (V + BT - 1) // BT - 1
    in_specs = [
        pl.BlockSpec((D, BT), lambda i, q=q: (0, jnp.minimum(i + q * NBQ, last_blk)))
        for q in range(4)
    ]
    table_pk = pl.pallas_call(
        _tr_body,
        grid=(NBQ,),
        in_specs=in_specs,
        out_specs=pl.BlockSpec((BT, 2 * D), lambda i: (i, 0)),
        out_shape=jax.ShapeDtypeStruct((OFF, 2 * D), jnp.int32),
    )(table_t, table_t, table_t, table_t)

    mesh = plsc.VectorSubcoreMesh(core_axis_name="c", subcore_axis_name="s")
    out2 = pl.kernel(
        _sc_body,
        out_type=jax.ShapeDtypeStruct((B // 2, 2 * D), jnp.float32),
        mesh=mesh,
        scratch_types=(
            [pltpu.VMEM((ROWS,), jnp.int32)] * 2
            + [pltpu.VMEM((ROWS,), jnp.int32)] * 2
            + [pltpu.VMEM((S * KP,), jnp.int32)] * 2
            + [pltpu.VMEM((S * KP,), jnp.float32)] * 2
            + [pltpu.VMEM((ROWS, 2 * D), jnp.int32)] * 2
            + [pltpu.VMEM((S // 2, 2 * D), jnp.float32)] * 2
            + [pltpu.SemaphoreType.DMA] * 2
        ),
    )(ids_flat, ids_pad, w_flat, table_pk)
    return out2.reshape(B, D)
